# Initial kernel scaffold; baseline (speedup 1.0000x reference)
import jax
import jax.numpy as jnp
from jax.experimental import pallas as pl


def kernel(h, e, edge_index, W_edge, b_edge, root, bias):
    raise NotImplementedError("write your pallas kernel here")


# speedup vs baseline: 3.9613x; 3.9613x over previous
import jax
import jax.numpy as jnp
from jax import lax
from jax.experimental import pallas as pl
from jax.experimental.pallas import tpu as pltpu
from jax.experimental.pallas import tpu_sc as plsc

N = 10000
E = 160000
F = 16
NC = 2
NS = 16
NW = NC * NS
EW = E // NW
KCH = 125
NCHUNK = EW // KCH
RPT = N // NS


def _sc_mesh():
    return plsc.VectorSubcoreMesh(core_axis_name="c", subcore_axis_name="s",
                                  num_cores=NC, num_subcores=NS)


def _gather_h_src(h, src_idx):
    def body(h_hbm, idx_hbm, out_hbm, idx_v, rows_v, sem):
        cid = lax.axis_index("c")
        sid = lax.axis_index("s")
        wid = sid * NC + cid
        pltpu.sync_copy(idx_hbm.at[wid], idx_v)

        def fire(j, carry):
            pltpu.async_copy(h_hbm.at[idx_v.at[j]],
                             rows_v.at[pl.ds(j * KCH, KCH)], sem)
            return carry

        lax.fori_loop(0, NCHUNK, fire, 0)

        def drain(j, carry):
            pltpu.make_async_copy(h_hbm.at[idx_v.at[j]],
                                  rows_v.at[pl.ds(j * KCH, KCH)], sem).wait()
            return carry

        lax.fori_loop(0, NCHUNK, drain, 0)
        pltpu.sync_copy(rows_v, out_hbm.at[pl.ds(wid * EW, EW)])

    return pl.kernel(
        body,
        out_type=jax.ShapeDtypeStruct((E, F), jnp.float32),
        mesh=_sc_mesh(),
        scratch_types=[
            pltpu.VMEM((NCHUNK, KCH), jnp.int32),
            pltpu.VMEM((EW, F), jnp.float32),
            pltpu.SemaphoreType.DMA,
        ],
        compiler_params=pltpu.CompilerParams(use_tc_tiling_on_sc=False),
    )(h, src_idx)


def _scatter_add(m, dst_idx):
    def body(m_hbm, idx_hbm, out_hbm, idx_v, mrows_v, zrows_v, acc_sh, sem):
        cid = lax.axis_index("c")
        sid = lax.axis_index("s")
        wid = sid * NC + cid

        def zrow(i, carry):
            zrows_v[i, :] = jnp.zeros((F,), jnp.float32)
            return carry

        lax.fori_loop(0, RPT, zrow, 0)
        pltpu.sync_copy(zrows_v, acc_sh.at[pl.ds(sid * RPT, RPT)])
        pltpu.sync_copy(idx_hbm.at[wid], idx_v)
        pltpu.async_copy(m_hbm.at[pl.ds(wid * EW, EW)], mrows_v, sem).wait()
        plsc.subcore_barrier()

        def scat(j, carry):
            pltpu.sync_copy(mrows_v.at[pl.ds(j * KCH, KCH)],
                            acc_sh.at[idx_v.at[j]], add=True)
            return carry

        lax.fori_loop(0, NCHUNK, scat, 0)
        plsc.subcore_barrier()
        pltpu.sync_copy(acc_sh.at[pl.ds(sid * RPT, RPT)],
                        out_hbm.at[cid, pl.ds(sid * RPT, RPT)])

    return pl.kernel(
        body,
        out_type=jax.ShapeDtypeStruct((NC, N, F), jnp.float32),
        mesh=_sc_mesh(),
        scratch_types=[
            pltpu.VMEM((NCHUNK, KCH), jnp.int32),
            pltpu.VMEM((EW, F), jnp.float32),
            pltpu.VMEM((RPT, F), jnp.float32),
            pltpu.VMEM_SHARED((N, F), jnp.float32),
            pltpu.SemaphoreType.DMA,
        ],
        compiler_params=pltpu.CompilerParams(use_tc_tiling_on_sc=False),
    )(m, dst_idx)


def _edge_messages(e_arr, h_src, Wz, Bm):
    TE = 8000
    FF = F * F

    def body(e_ref, hs_ref, wz_ref, bm_ref, o_ref):
        e_t = e_ref[...].astype(jnp.bfloat16)
        hs = hs_ref[...].astype(jnp.bfloat16)
        qq = lax.broadcasted_iota(jnp.int32, (F, FF), 1)
        dd = lax.broadcasted_iota(jnp.int32, (F, FF), 0)
        rep = (qq // F == dd).astype(jnp.bfloat16)
        til = (qq % F == dd).astype(jnp.bfloat16)
        eb = jnp.dot(e_t, rep,
                     preferred_element_type=jnp.float32).astype(jnp.bfloat16)
        hb = jnp.dot(hs, til,
                     preferred_element_type=jnp.float32).astype(jnp.bfloat16)
        z = eb * hb
        mm = jnp.dot(z, wz_ref[...].astype(jnp.bfloat16),
                     preferred_element_type=jnp.float32)
        mm = mm + jnp.dot(hs, bm_ref[...].astype(jnp.bfloat16),
                          preferred_element_type=jnp.float32)
        o_ref[...] = mm

    return pl.pallas_call(
        body,
        grid=(E // TE,),
        in_specs=[
            pl.BlockSpec((TE, F), lambda i: (i, 0)),
            pl.BlockSpec((TE, F), lambda i: (i, 0)),
            pl.BlockSpec((FF, F), lambda i: (0, 0)),
            pl.BlockSpec((F, F), lambda i: (0, 0)),
        ],
        out_specs=pl.BlockSpec((TE, F), lambda i: (i, 0)),
        out_shape=jax.ShapeDtypeStruct((E, F), jnp.float32),
    )(e_arr, h_src, Wz, Bm)


def _finalize(h, root, bias_row, p0, p1):
    def body(h_ref, r_ref, b_ref, p0_ref, p1_ref, o_ref):
        o_ref[...] = (jnp.dot(h_ref[...], r_ref[...],
                              preferred_element_type=jnp.float32)
                      + b_ref[...] + p0_ref[...] + p1_ref[...])

    return pl.pallas_call(
        body,
        out_shape=jax.ShapeDtypeStruct((N, F), jnp.float32),
    )(h, root, bias_row, p0, p1)


def kernel(h, e, edge_index, W_edge, b_edge, root, bias):
    src = edge_index[0].reshape(NW, NCHUNK, KCH)
    dst = edge_index[1].reshape(NW, NCHUNK, KCH)
    h_src = _gather_h_src(h, src)
    Wz = W_edge.reshape(F * F, F)
    Bm = b_edge.reshape(F, F)
    m = _edge_messages(e, h_src, Wz, Bm)
    partials = _scatter_add(m, dst)
    return _finalize(h, root, bias.reshape(1, F), partials[0], partials[1])


# speedup vs baseline: 4.7101x; 1.1890x over previous
import jax
import jax.numpy as jnp
from jax import lax
from jax.experimental import pallas as pl
from jax.experimental.pallas import tpu as pltpu
from jax.experimental.pallas import tpu_sc as plsc

N = 10000
E = 160000
F = 16
NC = 2
NS = 16
NW = NC * NS
EW = E // NW
KCH = 125
NCHUNK = EW // KCH
RPT = N // NS


def _sc_mesh():
    return plsc.VectorSubcoreMesh(core_axis_name="c", subcore_axis_name="s",
                                  num_cores=NC, num_subcores=NS)


def _gather_h_src(h, src_idx):
    def body(h_hbm, idx_hbm, out_hbm, idx_v, rows_v, sem):
        cid = lax.axis_index("c")
        sid = lax.axis_index("s")
        wid = sid * NC + cid
        pltpu.sync_copy(idx_hbm.at[wid], idx_v)

        def fire(j, carry):
            pltpu.async_copy(h_hbm.at[idx_v.at[j]],
                             rows_v.at[pl.ds(j * KCH, KCH)], sem)
            return carry

        lax.fori_loop(0, NCHUNK, fire, 0)

        def drain(j, carry):
            pltpu.make_async_copy(h_hbm.at[idx_v.at[j]],
                                  rows_v.at[pl.ds(j * KCH, KCH)], sem).wait()
            return carry

        lax.fori_loop(0, NCHUNK, drain, 0)
        pltpu.sync_copy(rows_v, out_hbm.at[pl.ds(wid * EW, EW)])

    return pl.kernel(
        body,
        out_type=jax.ShapeDtypeStruct((E, F), jnp.float32),
        mesh=_sc_mesh(),
        scratch_types=[
            pltpu.VMEM((NCHUNK, KCH), jnp.int32),
            pltpu.VMEM((EW, F), jnp.float32),
            pltpu.SemaphoreType.DMA,
        ],
        compiler_params=pltpu.CompilerParams(use_tc_tiling_on_sc=False),
    )(h, src_idx)


def _scatter_add(m, dst_idx):
    def body(m_hbm, idx_hbm, out_hbm, idx_v, mrows_v, zrows_v, acc_sh, sem):
        cid = lax.axis_index("c")
        sid = lax.axis_index("s")
        wid = sid * NC + cid

        def zrow(i, carry):
            zrows_v[i, :] = jnp.zeros((F,), jnp.float32)
            return carry

        lax.fori_loop(0, RPT, zrow, 0)
        pltpu.sync_copy(zrows_v, acc_sh.at[pl.ds(sid * RPT, RPT)])
        pltpu.sync_copy(idx_hbm.at[wid], idx_v)
        pltpu.async_copy(m_hbm.at[pl.ds(wid * EW, EW)], mrows_v, sem).wait()
        plsc.subcore_barrier()

        def scat(j, carry):
            pltpu.sync_copy(mrows_v.at[pl.ds(j * KCH, KCH)],
                            acc_sh.at[idx_v.at[j]], add=True)
            return carry

        lax.fori_loop(0, NCHUNK, scat, 0)
        plsc.subcore_barrier()
        pltpu.sync_copy(acc_sh.at[pl.ds(sid * RPT, RPT)],
                        out_hbm.at[cid, pl.ds(sid * RPT, RPT)])

    return pl.kernel(
        body,
        out_type=jax.ShapeDtypeStruct((NC, N, F), jnp.float32),
        mesh=_sc_mesh(),
        scratch_types=[
            pltpu.VMEM((NCHUNK, KCH), jnp.int32),
            pltpu.VMEM((EW, F), jnp.float32),
            pltpu.VMEM((RPT, F), jnp.float32),
            pltpu.VMEM_SHARED((N, F), jnp.float32),
            pltpu.SemaphoreType.DMA,
        ],
        compiler_params=pltpu.CompilerParams(use_tc_tiling_on_sc=False),
    )(m, dst_idx)


def _edge_messages_t(eT_arr, hsT_arr, WzT, BmT):
    TE = 16000
    FF = F * F

    def body(e_ref, hs_ref, wz_ref, bm_ref, o_ref):
        eT = e_ref[...].astype(jnp.bfloat16)
        hsT = hs_ref[...].astype(jnp.bfloat16)
        ebT = jnp.concatenate(
            [jnp.broadcast_to(eT[d:d + 1, :], (F, TE)) for d in range(F)],
            axis=0)
        hbT = jnp.concatenate([hsT] * F, axis=0)
        zT = ebT * hbT
        mT = jnp.dot(wz_ref[...].astype(jnp.bfloat16), zT,
                     preferred_element_type=jnp.float32)
        mT = mT + jnp.dot(bm_ref[...].astype(jnp.bfloat16), hsT,
                          preferred_element_type=jnp.float32)
        o_ref[...] = mT

    return pl.pallas_call(
        body,
        grid=(E // TE,),
        in_specs=[
            pl.BlockSpec((F, TE), lambda i: (0, i)),
            pl.BlockSpec((F, TE), lambda i: (0, i)),
            pl.BlockSpec((F, FF), lambda i: (0, 0)),
            pl.BlockSpec((F, F), lambda i: (0, 0)),
        ],
        out_specs=pl.BlockSpec((F, TE), lambda i: (0, i)),
        out_shape=jax.ShapeDtypeStruct((F, E), jnp.float32),
    )(eT_arr, hsT_arr, WzT, BmT)


def _finalize(h, root, bias_row, p0, p1):
    def body(h_ref, r_ref, b_ref, p0_ref, p1_ref, o_ref):
        o_ref[...] = (jnp.dot(h_ref[...], r_ref[...],
                              preferred_element_type=jnp.float32)
                      + b_ref[...] + p0_ref[...] + p1_ref[...])

    return pl.pallas_call(
        body,
        out_shape=jax.ShapeDtypeStruct((N, F), jnp.float32),
    )(h, root, bias_row, p0, p1)


def kernel(h, e, edge_index, W_edge, b_edge, root, bias):
    src = edge_index[0].reshape(NW, NCHUNK, KCH)
    dst = edge_index[1].reshape(NW, NCHUNK, KCH)
    h_src = _gather_h_src(h, src)
    WzT = jnp.transpose(W_edge.reshape(F * F, F))
    BmT = jnp.transpose(b_edge.reshape(F, F))
    mT = _edge_messages_t(jnp.transpose(e), jnp.transpose(h_src), WzT, BmT)
    m = jnp.transpose(mT)
    partials = _scatter_add(m, dst)
    return _finalize(h, root, bias.reshape(1, F), partials[0], partials[1])
